# baseline (speedup 1.0000x reference)
import functools

import jax
import jax.numpy as jnp
from jax import lax
from jax.experimental import pallas as pl
from jax.experimental.pallas import tpu as pltpu
from jax.experimental.pallas import tpu_sc as plsc

TOP_K = 2
CAPACITY_FACTOR = 1.25


def _cumsum_excl(v):
    T, E = v.shape
    out = v
    d = 1
    while d < T:
        shifted = jnp.concatenate(
            [jnp.zeros((d, E), v.dtype), out[: T - d, :]], axis=0
        )
        out = out + shifted
        d *= 2
    return out - v


def _route_body(
    xf_ref, WrT_ref, br_ref, slots_ref, w_ref, cnt_ref, *, T, E, C, DUMMY
):
    logits = (
        jnp.dot(
            xf_ref[...], WrT_ref[...].T,
            preferred_element_type=jnp.float32,
        )
        + br_ref[...]
    )
    m = jnp.max(logits, axis=1, keepdims=True)
    p = jnp.exp(logits - m)
    iota = lax.broadcasted_iota(jnp.int32, (T, E), 1)
    p1 = jnp.max(p, axis=1, keepdims=True)
    idx1 = jnp.min(jnp.where(p == p1, iota, E), axis=1, keepdims=True)
    oh1 = iota == idx1
    pm = jnp.where(oh1, -1.0, p)
    p2 = jnp.max(pm, axis=1, keepdims=True)
    idx2 = jnp.min(jnp.where(pm == p2, iota, E), axis=1, keepdims=True)
    oh2 = iota == idx2
    oh1f = oh1.astype(jnp.float32)
    oh2f = oh2.astype(jnp.float32)
    den = p1 + p2
    g1 = p1 / den
    g2 = p2 / den
    c1 = _cumsum_excl(oh1f)
    c2 = _cumsum_excl(oh2f)
    tot1 = jnp.sum(oh1f, axis=0, keepdims=True)
    tot2 = jnp.sum(oh2f, axis=0, keepdims=True)
    pos0 = jnp.sum(c1 * oh1f, axis=1, keepdims=True).astype(jnp.int32)
    pos1 = jnp.sum((c2 + tot1) * oh2f, axis=1, keepdims=True).astype(
        jnp.int32
    )
    v0 = pos0 < C
    v1 = pos1 < C
    slot0 = jnp.where(v0, idx1 * C + pos0, DUMMY)
    slot1 = jnp.where(v1, idx2 * C + pos1, DUMMY)
    slots_ref[...] = jnp.concatenate([slot0, slot1], axis=1)
    cnt_ref[...] = jnp.minimum(tot1 + tot2, float(C)).astype(jnp.int32)
    a0 = g1 * v0.astype(jnp.float32)
    a1 = g2 * v1.astype(jnp.float32)
    iota_c = lax.broadcasted_iota(jnp.int32, (T, C), 1)
    m0 = (iota_c == pos0).astype(jnp.float32) * a0
    m1 = (iota_c == pos1).astype(jnp.float32) * a1
    dn = (((0,), (0,)), ((), ()))
    w = lax.dot_general(
        oh1f, m0, dn, preferred_element_type=jnp.float32
    ) + lax.dot_general(oh2f, m1, dn, preferred_element_type=jnp.float32)
    w_ref[...] = w[:, None, :]


def _ffn_body(
    cnt_ref, buf_ref, w_ref, W1_ref, b1_ref, W2_ref, b2_ref, WcT_ref,
    bc_ref, out_ref, acc_ref, *, E, C, S, NF, FB,
):
    e = pl.program_id(0)
    f = pl.program_id(1)

    @pl.when((e == 0) & (f == 0))
    def _():
        acc_ref[...] = jnp.zeros_like(acc_ref)

    cnt_e = cnt_ref[0, e]
    row_valid = lax.broadcasted_iota(jnp.int32, (C, 1), 0) < cnt_e
    h = jnp.maximum(
        jnp.dot(buf_ref[...], W1_ref[0], preferred_element_type=jnp.float32)
        + b1_ref[pl.ds(e, 1), pl.ds(f * FB, FB)],
        0.0,
    )
    h = jnp.where(row_valid, h, 0.0)
    v = jnp.dot(w_ref[0], h, preferred_element_type=jnp.float32)
    acc_ref[...] += jnp.dot(
        v, W2_ref[0], preferred_element_type=jnp.float32
    )

    @pl.when(f == 0)
    def _():
        s_e = jnp.sum(w_ref[0])
        acc_ref[...] += s_e * b2_ref[pl.ds(e, 1), :]

    @pl.when((e == E - 1) & (f == NF - 1))
    def _():
        out_ref[...] = (
            lax.dot_general(
                acc_ref[...] / float(S), WcT_ref[...],
                (((1,), (1,)), ((), ())),
                preferred_element_type=jnp.float32,
            )
            + bc_ref[...]
        )


def _dispatch_rows(xf, slots_i, nrows):
    T, D = xf.shape
    info = plsc.get_sparse_core_info()
    nw = info.num_cores * info.num_subcores
    t_per_w = T // nw
    mesh = plsc.VectorSubcoreMesh(
        core_axis_name="c", subcore_axis_name="s"
    )

    @functools.partial(
        pl.kernel,
        out_type=jax.ShapeDtypeStruct((nrows, D), jnp.float32),
        mesh=mesh,
        scratch_types=[
            pltpu.VMEM((t_per_w,), jnp.int32),
            pltpu.VMEM((t_per_w,), jnp.int32),
            pltpu.VMEM((t_per_w, D), jnp.float32),
            pltpu.SemaphoreType.DMA,
            pltpu.SemaphoreType.DMA,
            pltpu.SemaphoreType.DMA,
        ],
    )
    def scatter_kernel(xf_hbm, slots_hbm, buf_hbm, idx0_v, idx1_v,
                       rows_v, sem_r, sem0, sem1):
        wid = lax.axis_index("s") * info.num_cores + lax.axis_index("c")
        base = wid * t_per_w
        rows_c = pltpu.async_copy(
            xf_hbm.at[pl.ds(base, t_per_w)], rows_v, sem_r
        )
        pltpu.sync_copy(slots_hbm.at[pl.ds(base, t_per_w)], idx0_v)
        pltpu.sync_copy(slots_hbm.at[pl.ds(T + base, t_per_w)], idx1_v)
        rows_c.wait()
        c0 = pltpu.async_copy(rows_v, buf_hbm.at[idx0_v], sem0)
        c1 = pltpu.async_copy(rows_v, buf_hbm.at[idx1_v], sem1)
        c0.wait()
        c1.wait()

    return scatter_kernel(xf, slots_i)


def kernel(x, Wr, br, W1, b1, W2, b2, Wc, bc):
    B, S, D = x.shape
    E, _, F = W1.shape
    NC = Wc.shape[1]
    T = B * S
    C = int(T * TOP_K / E * CAPACITY_FACTOR)
    NROWS = E * C + 8
    xf = x.reshape(T, D)

    slots_tk, w_mat, cnt = pl.pallas_call(
        functools.partial(_route_body, T=T, E=E, C=C, DUMMY=E * C),
        in_specs=[
            pl.BlockSpec((T, D), lambda: (0, 0)),
            pl.BlockSpec((E, D), lambda: (0, 0)),
            pl.BlockSpec((1, E), lambda: (0, 0)),
        ],
        out_specs=[
            pl.BlockSpec((T, 2), lambda: (0, 0)),
            pl.BlockSpec((E, 1, C), lambda: (0, 0, 0)),
            pl.BlockSpec((1, E), lambda: (0, 0)),
        ],
        out_shape=[
            jax.ShapeDtypeStruct((T, 2), jnp.int32),
            jax.ShapeDtypeStruct((E, 1, C), jnp.float32),
            jax.ShapeDtypeStruct((1, E), jnp.int32),
        ],
    )(xf, Wr.T, br.reshape(1, E))

    slots_i = slots_tk.T.reshape(2 * T)
    buf = _dispatch_rows(xf, slots_i, NROWS)

    NF = 2
    FB = F // NF
    out = pl.pallas_call(
        functools.partial(_ffn_body, E=E, C=C, S=S, NF=NF, FB=FB),
        grid=(E, NF),
        in_specs=[
            pl.BlockSpec(memory_space=pltpu.SMEM),
            pl.BlockSpec((C, D), lambda e, f: (e, 0)),
            pl.BlockSpec((1, 1, C), lambda e, f: (e, 0, 0)),
            pl.BlockSpec((1, D, FB), lambda e, f: (e, 0, f)),
            pl.BlockSpec((E, F), lambda e, f: (0, 0)),
            pl.BlockSpec((1, FB, D), lambda e, f: (e, f, 0)),
            pl.BlockSpec((E, D), lambda e, f: (0, 0)),
            pl.BlockSpec((NC, D), lambda e, f: (0, 0)),
            pl.BlockSpec((1, NC), lambda e, f: (0, 0)),
        ],
        out_specs=pl.BlockSpec((1, NC), lambda e, f: (0, 0)),
        out_shape=jax.ShapeDtypeStruct((1, NC), jnp.float32),
        scratch_shapes=[pltpu.VMEM((1, D), jnp.float32)],
        compiler_params=pltpu.CompilerParams(
            dimension_semantics=("arbitrary", "arbitrary")
        ),
    )(
        cnt,
        buf,
        w_mat,
        W1,
        b1,
        W2,
        b2,
        Wc.T,
        bc.reshape(1, NC),
    )
    return out.reshape(B, NC)


# speedup vs baseline: 1.0365x; 1.0365x over previous
import functools

import jax
import jax.numpy as jnp
from jax import lax
from jax.experimental import pallas as pl
from jax.experimental.pallas import tpu as pltpu
from jax.experimental.pallas import tpu_sc as plsc

TOP_K = 2
CAPACITY_FACTOR = 1.25


def _cumsum_excl(v):
    T, E = v.shape
    out = v
    d = 1
    while d < T:
        shifted = jnp.concatenate(
            [jnp.zeros((d, E), v.dtype), out[: T - d, :]], axis=0
        )
        out = out + shifted
        d *= 2
    return out - v


def _route_body(
    xf_ref, WrT_ref, br_ref, slots_ref, w_ref, cnt_ref, *, T, E, C, DUMMY
):
    logits = (
        jnp.dot(
            xf_ref[...], WrT_ref[...].T,
            preferred_element_type=jnp.float32,
        )
        + br_ref[...]
    )
    m = jnp.max(logits, axis=1, keepdims=True)
    p = jnp.exp(logits - m)
    iota = lax.broadcasted_iota(jnp.int32, (T, E), 1)
    p1 = jnp.max(p, axis=1, keepdims=True)
    idx1 = jnp.min(jnp.where(p == p1, iota, E), axis=1, keepdims=True)
    oh1 = iota == idx1
    pm = jnp.where(oh1, -1.0, p)
    p2 = jnp.max(pm, axis=1, keepdims=True)
    idx2 = jnp.min(jnp.where(pm == p2, iota, E), axis=1, keepdims=True)
    oh2 = iota == idx2
    oh1f = oh1.astype(jnp.float32)
    oh2f = oh2.astype(jnp.float32)
    den = p1 + p2
    g1 = p1 / den
    g2 = p2 / den
    c1 = _cumsum_excl(oh1f)
    c2 = _cumsum_excl(oh2f)
    tot1 = jnp.sum(oh1f, axis=0, keepdims=True)
    tot2 = jnp.sum(oh2f, axis=0, keepdims=True)
    pos0 = jnp.sum(c1 * oh1f, axis=1, keepdims=True).astype(jnp.int32)
    pos1 = jnp.sum((c2 + tot1) * oh2f, axis=1, keepdims=True).astype(
        jnp.int32
    )
    v0 = pos0 < C
    v1 = pos1 < C
    slot0 = jnp.where(v0, idx1 * C + pos0, DUMMY)
    slot1 = jnp.where(v1, idx2 * C + pos1, DUMMY)
    slots_ref[...] = jnp.concatenate([slot0, slot1], axis=1)
    cnt_ref[...] = jnp.minimum(tot1 + tot2, float(C)).astype(jnp.int32)
    a0 = g1 * v0.astype(jnp.float32)
    a1 = g2 * v1.astype(jnp.float32)
    iota_c = lax.broadcasted_iota(jnp.int32, (T, C), 1)
    m0 = (iota_c == pos0).astype(jnp.float32) * a0
    m1 = (iota_c == pos1).astype(jnp.float32) * a1
    dn = (((0,), (0,)), ((), ()))
    w = lax.dot_general(
        oh1f, m0, dn, preferred_element_type=jnp.float32
    ) + lax.dot_general(oh2f, m1, dn, preferred_element_type=jnp.float32)
    w_ref[...] = w[:, None, :]


def _ffn_body(
    cnt_ref, buf_ref, w_ref, W1_ref, b1_ref, W2_ref, b2_ref, WcT_ref,
    bc_ref, out_ref, acc_ref, *, E, C, S,
):
    e = pl.program_id(0)

    @pl.when(e == 0)
    def _():
        acc_ref[...] = jnp.zeros_like(acc_ref)

    cnt_e = cnt_ref[0, e]
    row_valid = lax.broadcasted_iota(jnp.int32, (C, 1), 0) < cnt_e
    h = jnp.maximum(
        jnp.dot(buf_ref[...], W1_ref[0], preferred_element_type=jnp.float32)
        + b1_ref[pl.ds(e, 1), :],
        0.0,
    )
    h = jnp.where(row_valid, h, 0.0)
    v = jnp.dot(w_ref[0], h, preferred_element_type=jnp.float32)
    s_e = jnp.sum(w_ref[0])
    acc_ref[...] += (
        jnp.dot(v, W2_ref[0], preferred_element_type=jnp.float32)
        + s_e * b2_ref[pl.ds(e, 1), :]
    )

    @pl.when(e == E - 1)
    def _():
        out_ref[...] = (
            lax.dot_general(
                acc_ref[...] / float(S), WcT_ref[...],
                (((1,), (1,)), ((), ())),
                preferred_element_type=jnp.float32,
            )
            + bc_ref[...]
        )


def _dispatch_rows(xf, slots_i, nrows):
    T, D = xf.shape
    info = plsc.get_sparse_core_info()
    nw = info.num_cores * info.num_subcores
    t_per_w = T // nw
    mesh = plsc.VectorSubcoreMesh(
        core_axis_name="c", subcore_axis_name="s"
    )

    half = t_per_w // 2

    @functools.partial(
        pl.kernel,
        out_type=jax.ShapeDtypeStruct((nrows, D), jnp.float32),
        mesh=mesh,
        scratch_types=[
            pltpu.VMEM((half,), jnp.int32),
            pltpu.VMEM((half,), jnp.int32),
            pltpu.VMEM((half,), jnp.int32),
            pltpu.VMEM((half,), jnp.int32),
            pltpu.VMEM((half, D), jnp.float32),
            pltpu.VMEM((half, D), jnp.float32),
            pltpu.SemaphoreType.DMA,
            pltpu.SemaphoreType.DMA,
            pltpu.SemaphoreType.DMA,
            pltpu.SemaphoreType.DMA,
            pltpu.SemaphoreType.DMA,
            pltpu.SemaphoreType.DMA,
        ],
    )
    def scatter_kernel(xf_hbm, slots_hbm, buf_hbm, idx0a_v, idx0b_v,
                       idx1a_v, idx1b_v, rows_a, rows_b, sem_ra, sem_rb,
                       sem0, sem1, sem2, sem3):
        wid = lax.axis_index("s") * info.num_cores + lax.axis_index("c")
        base = wid * t_per_w
        in_a = pltpu.async_copy(xf_hbm.at[pl.ds(base, half)], rows_a,
                                sem_ra)
        in_b = pltpu.async_copy(xf_hbm.at[pl.ds(base + half, half)],
                                rows_b, sem_rb)
        pltpu.sync_copy(slots_hbm.at[pl.ds(base, half)], idx0a_v)
        pltpu.sync_copy(slots_hbm.at[pl.ds(base + half, half)], idx0b_v)
        pltpu.sync_copy(slots_hbm.at[pl.ds(T + base, half)], idx1a_v)
        pltpu.sync_copy(slots_hbm.at[pl.ds(T + base + half, half)],
                        idx1b_v)
        in_a.wait()
        c0 = pltpu.async_copy(rows_a, buf_hbm.at[idx0a_v], sem0)
        c1 = pltpu.async_copy(rows_a, buf_hbm.at[idx1a_v], sem1)
        in_b.wait()
        c2 = pltpu.async_copy(rows_b, buf_hbm.at[idx0b_v], sem2)
        c3 = pltpu.async_copy(rows_b, buf_hbm.at[idx1b_v], sem3)
        c0.wait()
        c1.wait()
        c2.wait()
        c3.wait()

    return scatter_kernel(xf, slots_i)


def kernel(x, Wr, br, W1, b1, W2, b2, Wc, bc):
    B, S, D = x.shape
    E, _, F = W1.shape
    NC = Wc.shape[1]
    T = B * S
    C = int(T * TOP_K / E * CAPACITY_FACTOR)
    NROWS = E * C + 8
    xf = x.reshape(T, D)

    slots_tk, w_mat, cnt = pl.pallas_call(
        functools.partial(_route_body, T=T, E=E, C=C, DUMMY=E * C),
        in_specs=[
            pl.BlockSpec((T, D), lambda: (0, 0)),
            pl.BlockSpec((E, D), lambda: (0, 0)),
            pl.BlockSpec((1, E), lambda: (0, 0)),
        ],
        out_specs=[
            pl.BlockSpec((T, 2), lambda: (0, 0)),
            pl.BlockSpec((E, 1, C), lambda: (0, 0, 0)),
            pl.BlockSpec((1, E), lambda: (0, 0)),
        ],
        out_shape=[
            jax.ShapeDtypeStruct((T, 2), jnp.int32),
            jax.ShapeDtypeStruct((E, 1, C), jnp.float32),
            jax.ShapeDtypeStruct((1, E), jnp.int32),
        ],
    )(xf, Wr.T, br.reshape(1, E))

    slots_i = slots_tk.T.reshape(2 * T)
    buf = _dispatch_rows(xf, slots_i, NROWS)

    out = pl.pallas_call(
        functools.partial(_ffn_body, E=E, C=C, S=S),
        grid=(E,),
        in_specs=[
            pl.BlockSpec(memory_space=pltpu.SMEM),
            pl.BlockSpec((C, D), lambda e: (e, 0)),
            pl.BlockSpec((1, 1, C), lambda e: (e, 0, 0)),
            pl.BlockSpec((1, D, F), lambda e: (e, 0, 0)),
            pl.BlockSpec((E, F), lambda e: (0, 0)),
            pl.BlockSpec((1, F, D), lambda e: (e, 0, 0)),
            pl.BlockSpec((E, D), lambda e: (0, 0)),
            pl.BlockSpec((NC, D), lambda e: (0, 0)),
            pl.BlockSpec((1, NC), lambda e: (0, 0)),
        ],
        out_specs=pl.BlockSpec((1, NC), lambda e: (0, 0)),
        out_shape=jax.ShapeDtypeStruct((1, NC), jnp.float32),
        scratch_shapes=[pltpu.VMEM((1, D), jnp.float32)],
        compiler_params=pltpu.CompilerParams(
            dimension_semantics=("arbitrary",)
        ),
    )(
        cnt,
        buf,
        w_mat,
        W1,
        b1,
        W2,
        b2,
        Wc.T,
        bc.reshape(1, NC),
    )
    return out.reshape(B, NC)
